# baseline (speedup 1.0000x reference)
import functools

import jax
import jax.numpy as jnp
from jax import lax
from jax.experimental import pallas as pl
from jax.experimental.pallas import tpu as pltpu
from jax.experimental.pallas import tpu_sc as plsc

B, C, H, W = 4, 384, 14, 14
IMG = 224
SCALE = IMG // H
P = 50
N = 50
ALPHA = 0.7
K = C // 2
HW = H * W

NC = 2
NS = 16
CPT = C // 8
KP_PAD = 208
FM_PAD = 256


def _sc_body(x_hbm, kp_hbm, outx_hbm, fm_hbm,
             xloc, kploc, pix_ref, wv_ref, scv, sc_all, keep_ref,
             fmloc, fm8, shared_sc, shared_fm):
    core = lax.axis_index("c")
    sub = lax.axis_index("s")
    b_local = sub // 8
    chunk = sub % 8
    b = core * 2 + b_local
    base_row = b * C + chunk * CPT

    pltpu.sync_copy(x_hbm.at[pl.ds(base_row, CPT)], xloc)
    pltpu.sync_copy(kp_hbm.at[b], kploc)

    iota = lax.iota(jnp.int32, 16)
    zf = jnp.zeros((16,), jnp.float32)
    zi = jnp.zeros((16,), jnp.int32)

    for t in range(8):
        grp = t // 4
        slot = iota + (t % 4) * 16
        eidx = jnp.minimum(2 * slot + grp * (2 * P), 2 * (P + N) - 2)
        r = plsc.load_gather(kploc, [eidx])
        cc = plsc.load_gather(kploc, [eidx + 1])
        pixv = (r // SCALE) * W + cc // SCALE
        valid = slot < (P if grp == 0 else N)
        wconst = ALPHA if grp == 0 else ALPHA - 1.0
        wv = jnp.where(valid, jnp.full((16,), wconst, jnp.float32), zf)
        pix_ref[pl.ds(t * 16, 16)] = pixv
        wv_ref[pl.ds(t * 16, 16)] = wv

    rows = [iota + 16 * i for i in range(3)]

    def score_body(k, carry):
        kk = jnp.full((16,), k, jnp.int32)
        pv = plsc.load_gather(pix_ref, [kk])
        wk = plsc.load_gather(wv_ref, [kk])
        return tuple(
            s + wk * plsc.load_gather(xloc, [rows[i], pv])
            for i, s in enumerate(carry)
        )

    s0, s1, s2 = lax.fori_loop(0, 128, score_body, (zf, zf, zf))
    scv[pl.ds(0, 16)] = s0
    scv[pl.ds(16, 16)] = s1
    scv[pl.ds(32, 16)] = s2
    pltpu.sync_copy(scv, shared_sc.at[pl.ds(b_local * C + chunk * CPT, CPT)])
    plsc.subcore_barrier()
    pltpu.sync_copy(shared_sc.at[pl.ds(b_local * C, C)], sc_all)

    myscores = (s0, s1, s2)
    cids = [iota + chunk * CPT + 16 * i for i in range(3)]

    def rank_body(j, carry):
        jj = jnp.full((16,), j, jnp.int32)
        sj = plsc.load_gather(sc_all, [jj])
        return tuple(
            r + ((sj < si) | ((sj == si) & (jj < ci))).astype(jnp.int32)
            for r, si, ci in zip(carry, myscores, cids)
        )

    ranks = lax.fori_loop(0, C, rank_body, (zi, zi, zi))

    for i, r in enumerate(ranks):
        keep_ref[pl.ds(16 * i, 16)] = jnp.where(r >= K, 1.0, 0.0).astype(
            jnp.float32)

    def mask_body(c_loc, _):
        ks = plsc.load_gather(keep_ref, [jnp.full((16,), c_loc, jnp.int32)])
        for v in range(13):
            off = 180 if v == 12 else v * 16
            xloc[c_loc, pl.ds(off, 16)] = xloc[c_loc, pl.ds(off, 16)] * ks
        return 0

    lax.fori_loop(0, CPT, mask_body, 0)
    pltpu.sync_copy(xloc, outx_hbm.at[pl.ds(base_row, CPT)])

    for v in range(FM_PAD // 16):
        fmloc[pl.ds(v * 16, 16)] = zi
    for r, ci in zip(ranks, cids):
        plsc.store_scatter(fmloc, [jnp.minimum(r, FM_PAD - 16)], ci + 1,
                           mask=r < K)
    pltpu.sync_copy(
        fmloc, shared_fm.at[pl.ds((b_local * 8 + chunk) * FM_PAD, FM_PAD)])
    plsc.subcore_barrier()

    @pl.when(chunk == 0)
    def _():
        pltpu.sync_copy(
            shared_fm.at[pl.ds(b_local * 8 * FM_PAD, 8 * FM_PAD)], fm8)
        for v in range(12):
            acc = fm8[pl.ds(v * 16, 16)]
            for t in range(1, 8):
                acc = acc + fm8[pl.ds(t * FM_PAD + v * 16, 16)]
            fmloc[pl.ds(v * 16, 16)] = acc - 1
        pltpu.sync_copy(fmloc, fm_hbm.at[b])


@jax.jit
def kernel(x, pos_keypoints, keypoints):
    x2d = x.reshape(B * C, HW)
    kp = jnp.concatenate(
        [pos_keypoints.reshape(B, 2 * P), keypoints.reshape(B, 2 * N)], axis=1)
    kp = jnp.pad(kp, ((0, 0), (0, KP_PAD - 2 * (P + N))))

    mesh = plsc.VectorSubcoreMesh(
        core_axis_name="c", subcore_axis_name="s",
        num_cores=NC, num_subcores=NS)
    run = functools.partial(
        pl.kernel,
        out_type=(
            jax.ShapeDtypeStruct((B * C, HW), jnp.float32),
            jax.ShapeDtypeStruct((B, FM_PAD), jnp.int32),
        ),
        mesh=mesh,
        compiler_params=pltpu.CompilerParams(needs_layout_passes=False),
        scratch_types=[
            pltpu.VMEM((CPT, HW), jnp.float32),
            pltpu.VMEM((KP_PAD,), jnp.int32),
            pltpu.VMEM((128,), jnp.int32),
            pltpu.VMEM((128,), jnp.float32),
            pltpu.VMEM((CPT,), jnp.float32),
            pltpu.VMEM((C,), jnp.float32),
            pltpu.VMEM((CPT,), jnp.float32),
            pltpu.VMEM((FM_PAD,), jnp.int32),
            pltpu.VMEM((8 * FM_PAD,), jnp.int32),
            pltpu.VMEM_SHARED((2 * C,), jnp.float32),
            pltpu.VMEM_SHARED((2 * 8 * FM_PAD,), jnp.int32),
        ],
    )(_sc_body)
    out2d, fm = run(x2d, kp)
    return out2d.reshape(B, C, H, W), fm[:, :K]
